# baseline (speedup 1.0000x reference)
import functools

import jax
import jax.numpy as jnp
from jax import lax
from jax.experimental import pallas as pl
from jax.experimental.pallas import tpu as pltpu
from jax.experimental.pallas import tpu_sc as plsc

_B, _N, _COORD, _FEAT, _MID = 4, 8192, 2, 16, 8
_K = 32
_NW = 32
_CPAD = 16


def _knn_body(q_ref, pT_ref, g16_ref, idxc_ref, idxf_ref,
              skey_ref, bits_ref, *, n_pts, cstride, K):
    b = pl.program_id(0)
    q = q_ref[0]
    p = pT_ref[0]
    TM = q.shape[0]
    N = n_pts
    NB = N // 16
    qn = jnp.sum(q * q, axis=1, keepdims=True)
    pn = jnp.sum(p * p, axis=0, keepdims=True)
    dot = lax.dot_general(q, p, (((1,), (0,)), ((), ())),
                          preferred_element_type=jnp.float32)
    d2 = (qn + pn) - 2.0 * dot
    fbits = lax.bitcast_convert_type(d2, jnp.int32)
    skey_ref[...] = jnp.where(fbits >= 0, fbits, fbits ^ jnp.int32(0x7FFFFFFF))
    skey0 = skey_ref[...]
    lo0 = jnp.min(skey0, axis=1, keepdims=True)
    hi0 = jnp.max(skey0, axis=1, keepdims=True)
    niota = lax.broadcasted_iota(jnp.int32, (TM, N), 1)

    def bstep(_, c):
        lo, hi = c
        mid = lo + ((hi - lo) >> 1)
        s = skey_ref[...]
        cnt = jnp.sum(jnp.where(s <= mid, 1, 0), axis=1, keepdims=True)
        ge = cnt >= K
        return jnp.where(ge, lo, mid + 1), jnp.where(ge, mid, hi)

    _, t = lax.fori_loop(0, 32, bstep, (lo0, hi0))

    s = skey_ref[...]
    c_less = jnp.sum(jnp.where(s < t, 1, 0), axis=1, keepdims=True)
    c_le = jnp.sum(jnp.where(s <= t, 1, 0), axis=1, keepdims=True)
    r = K - c_less

    def tie_fn(_):
        def jstep(_, c):
            lo, hi = c
            mid = lo + ((hi - lo) >> 1)
            s2 = skey_ref[...]
            cc = jnp.sum(jnp.where((s2 == t) & (niota <= mid), 1, 0),
                         axis=1, keepdims=True)
            ge = cc >= r
            return jnp.where(ge, lo, mid + 1), jnp.where(ge, mid, hi)
        return lax.fori_loop(0, 13, jstep,
                             (jnp.zeros((TM, 1), jnp.int32),
                              jnp.full((TM, 1), N - 1, jnp.int32)))[1]

    trigger = jnp.max(c_le) > K
    jstar = lax.cond(trigger, tie_fn,
                     lambda _: jnp.full((TM, 1), N, jnp.int32), 0)

    s = skey_ref[...]
    mask = (s < t) | ((s == t) & (niota <= jstar))
    pow2row = (1 << (lax.broadcasted_iota(jnp.int32, (1, N), 1) & 15)
               ).astype(jnp.float32)
    contrib = jnp.where(mask, pow2row, 0.0)
    bitsf = lax.dot_general(contrib.astype(jnp.bfloat16), g16_ref[...],
                            (((1,), (0,)), ((), ())),
                            preferred_element_type=jnp.float32)
    bits_ref[...] = bitsf.astype(jnp.int32)

    jiota = lax.broadcasted_iota(jnp.int32, (TM, NB), 1)
    kiota = lax.broadcasted_iota(jnp.int32, (TM, K), 1)

    def estep(k, idxl):
        bts = bits_ref[...]
        posj = jnp.min(jnp.where(bts > 0, jiota, NB), axis=1, keepdims=True)
        wsel = jnp.max(jnp.where(jiota == posj, bts, 0), axis=1, keepdims=True)
        low = wsel & (0 - wsel)
        bitidx = lax.population_count(low - 1)
        n = posj * 16 + bitidx
        bits_ref[...] = jnp.where(jiota == posj, wsel & (wsel - 1), bts)
        return jnp.where(kiota == k, n, idxl)

    idxl = lax.fori_loop(0, K, estep, jnp.zeros((TM, K), jnp.int32))
    idxc_ref[0] = idxl + b * cstride
    idxf_ref[0] = idxl + b * n_pts


def _knn_pallas(coords, coordsT, M, N, TM):
    B = coords.shape[0]
    g16 = (
        (jnp.arange(N, dtype=jnp.int32)[:, None] >> 4)
        == jnp.arange(N // 16, dtype=jnp.int32)[None, :]
    ).astype(jnp.bfloat16)
    kern = functools.partial(_knn_body, n_pts=N, cstride=_N, K=_K)
    return pl.pallas_call(
        kern,
        grid=(B, M // TM),
        in_specs=[
            pl.BlockSpec((1, TM, _COORD), lambda b, mi: (b, mi, 0)),
            pl.BlockSpec((1, _COORD, N), lambda b, mi: (b, 0, 0)),
            pl.BlockSpec((N, N // 16), lambda b, mi: (0, 0)),
        ],
        out_specs=[
            pl.BlockSpec((1, TM, _K), lambda b, mi: (b, mi, 0)),
            pl.BlockSpec((1, TM, _K), lambda b, mi: (b, mi, 0)),
        ],
        out_shape=[
            jax.ShapeDtypeStruct((B, M, _K), jnp.int32),
            jax.ShapeDtypeStruct((B, M, _K), jnp.int32),
        ],
        scratch_shapes=[
            pltpu.VMEM((TM, N), jnp.int32),
            pltpu.VMEM((TM, N // 16), jnp.int32),
        ],
    )(coords, coordsT, g16)


def _make_sc_gather(V, D, R):
    bpw = R // _NW
    mesh = plsc.VectorSubcoreMesh(core_axis_name="c", subcore_axis_name="s")

    @functools.partial(
        pl.kernel, mesh=mesh,
        out_type=jax.ShapeDtypeStruct((R, D), jnp.float32),
        compiler_params=pltpu.CompilerParams(use_tc_tiling_on_sc=False),
        scratch_types=[
            pltpu.VMEM((bpw,), jnp.int32),
            pltpu.VMEM((bpw, D), jnp.float32),
            pltpu.SemaphoreType.DMA,
        ],
    )
    def gk(table_hbm, idx_hbm, out_hbm, idx_v, rows_v, sem):
        wid = lax.axis_index("s") * 2 + lax.axis_index("c")
        base = wid * bpw
        pltpu.sync_copy(idx_hbm.at[pl.ds(base, bpw)], idx_v)
        pltpu.async_copy(table_hbm.at[idx_v], rows_v, sem).wait()
        pltpu.sync_copy(rows_v, out_hbm.at[pl.ds(base, bpw)])

    return gk


def _sc_gather(table, idx):
    V, D = table.shape
    R = idx.shape[0]
    return _make_sc_gather(V, D, R)(table, idx)


def _dense_body(gc_ref, gf_ref, q_ref, w1_ref, b1_ref, w2_ref, b2_ref,
                lw_ref, lb_ref, out_ref, *, K, C, TM):
    q = q_ref[0]
    w1 = w1_ref[...]
    b1 = b1_ref[...]
    w2 = w2_ref[...]
    b2 = b2_ref[...]
    D = _MID
    CD = C * D
    riota_c = lax.broadcasted_iota(jnp.int32, (C, CD), 0)
    ciota_c = lax.broadcasted_iota(jnp.int32, (C, CD), 1)
    E1 = (ciota_c // D == riota_c).astype(jnp.float32)
    riota_d = lax.broadcasted_iota(jnp.int32, (D, CD), 0)
    ciota_d = lax.broadcasted_iota(jnp.int32, (D, CD), 1)
    E2 = (ciota_d % D == riota_d).astype(jnp.float32)

    dn = (((1,), (0,)), ((), ()))
    agg = jnp.zeros((TM, CD), jnp.float32)
    gc = gc_ref[0]
    gf = gf_ref[0]
    for k in range(K):
        rel = gc[:, k * _CPAD:k * _CPAD + 2] - q
        h = lax.dot_general(rel, w1, dn, preferred_element_type=jnp.float32) + b1
        h = jnp.maximum(h, 0.0)
        wk = lax.dot_general(h, w2, dn, preferred_element_type=jnp.float32) + b2
        nbk = gf[:, k * C:(k + 1) * C]
        x1 = lax.dot_general(nbk, E1, dn, preferred_element_type=jnp.float32)
        x2 = lax.dot_general(wk, E2, dn, preferred_element_type=jnp.float32)
        agg = agg + x1 * x2
    out = lax.dot_general(agg, lw_ref[...], dn,
                          preferred_element_type=jnp.float32) + lb_ref[...]
    out_ref[0] = jnp.where(out >= 0.0, out, 0.2 * out)


def _dense_pallas(gc, gf, coords, w1, b1, w2, b2, lw, lb, M, C, cout, TM):
    B = coords.shape[0]
    kern = functools.partial(_dense_body, K=_K, C=C, TM=TM)
    return pl.pallas_call(
        kern,
        grid=(B, M // TM),
        in_specs=[
            pl.BlockSpec((1, TM, _K * _CPAD), lambda b, mi: (b, mi, 0)),
            pl.BlockSpec((1, TM, _K * C), lambda b, mi: (b, mi, 0)),
            pl.BlockSpec((1, TM, _COORD), lambda b, mi: (b, mi, 0)),
            pl.BlockSpec((2, _MID), lambda b, mi: (0, 0)),
            pl.BlockSpec((1, _MID), lambda b, mi: (0, 0)),
            pl.BlockSpec((_MID, _MID), lambda b, mi: (0, 0)),
            pl.BlockSpec((1, _MID), lambda b, mi: (0, 0)),
            pl.BlockSpec((C * _MID, cout), lambda b, mi: (0, 0)),
            pl.BlockSpec((1, cout), lambda b, mi: (0, 0)),
        ],
        out_specs=pl.BlockSpec((1, TM, cout), lambda b, mi: (b, mi, 0)),
        out_shape=jax.ShapeDtypeStruct((B, M, cout), jnp.float32),
    )(gc, gf, coords, w1, b1.reshape(1, -1), w2, b2.reshape(1, -1),
      lw, lb.reshape(1, -1))


def _head_body(x_ref, w1_ref, b1_ref, w2_ref, b2_ref, o_ref):
    dn = (((1,), (0,)), ((), ()))
    h = lax.dot_general(x_ref[...], w1_ref[...], dn,
                        preferred_element_type=jnp.float32) + b1_ref[...]
    h = jnp.where(h >= 0.0, h, 0.2 * h)
    o_ref[...] = lax.dot_general(h, w2_ref[...], dn,
                                 preferred_element_type=jnp.float32) + b2_ref[...]


def _head_pallas(x, fc1_w, fc1_b, fc2_w, fc2_b):
    return pl.pallas_call(
        _head_body,
        out_shape=jax.ShapeDtypeStruct((x.shape[0], fc2_w.shape[1]), jnp.float32),
    )(x, fc1_w, fc1_b.reshape(1, -1), fc2_w, fc2_b.reshape(1, -1))


def kernel(coordinates, features,
           wn1_w0, wn1_b0, wn2_w0, wn2_b0, lin_w0, lin_b0,
           wn1_w1, wn1_b1, wn2_w1, wn2_b1, lin_w1, lin_b1,
           wn1_w2, wn1_b2, wn2_w2, wn2_b2, lin_w2, lin_b2,
           fc1_w, fc1_b, fc2_w, fc2_b):
    B = coordinates.shape[0]
    coordsT = jnp.transpose(coordinates, (0, 2, 1))
    cpad = jnp.pad(coordinates, ((0, 0), (0, 0), (0, _CPAD - _COORD)))
    cpad = cpad.reshape(B * _N, _CPAD)

    idxc0, idxf0 = _knn_pallas(coordinates, coordsT, M=1024, N=8192, TM=256)
    idxc1, idxf1 = _knn_pallas(coordinates, coordsT, M=256, N=1024, TM=256)
    idxc2, idxf2 = _knn_pallas(coordinates, coordsT, M=64, N=256, TM=64)

    idxc_all = jnp.concatenate(
        [idxc0.reshape(-1), idxc1.reshape(-1), idxc2.reshape(-1)])
    gc_all = _sc_gather(cpad, idxc_all)
    n0, n1 = 1024 * _K * B, 256 * _K * B
    gc0 = gc_all[:n0]
    gc1 = gc_all[n0:n0 + n1]
    gc2 = gc_all[n0 + n1:]

    gf0 = _sc_gather(features.reshape(B * _N, _FEAT), idxf0.reshape(-1))
    f1 = _dense_pallas(gc0.reshape(B, 1024, _K * _CPAD),
                       gf0.reshape(B, 1024, _K * _FEAT),
                       coordinates, wn1_w0, wn1_b0, wn2_w0, wn2_b0,
                       lin_w0, lin_b0, M=1024, C=_FEAT, cout=32, TM=256)

    gf1 = _sc_gather(f1.reshape(B * 1024, 32), idxf1.reshape(-1))
    f2 = _dense_pallas(gc1.reshape(B, 256, _K * _CPAD),
                       gf1.reshape(B, 256, _K * 32),
                       coordinates, wn1_w1, wn1_b1, wn2_w1, wn2_b1,
                       lin_w1, lin_b1, M=256, C=32, cout=64, TM=256)

    gf2 = _sc_gather(f2.reshape(B * 256, 64), idxf2.reshape(-1))
    f3 = _dense_pallas(gc2.reshape(B, 64, _K * _CPAD),
                       gf2.reshape(B, 64, _K * 64),
                       coordinates, wn1_w2, wn1_b2, wn2_w2, wn2_b2,
                       lin_w2, lin_b2, M=64, C=64, cout=128, TM=64)

    return _head_pallas(f3.reshape(B, 64 * 128), fc1_w, fc1_b, fc2_w, fc2_b)


# speedup vs baseline: 1.1602x; 1.1602x over previous
import functools

import jax
import jax.numpy as jnp
from jax import lax
from jax.experimental import pallas as pl
from jax.experimental.pallas import tpu as pltpu
from jax.experimental.pallas import tpu_sc as plsc

_B, _N, _COORD, _FEAT, _MID = 4, 8192, 2, 16, 8
_K = 32
_NW = 32
_CPAD = 16


def _knn_body(q_ref, pT_ref, g16_ref, idxc_ref, idxf_ref,
              skey_ref, bits_ref, *, n_pts, cstride, K):
    b = pl.program_id(0)
    q = q_ref[0]
    p = pT_ref[0]
    TM = q.shape[0]
    N = n_pts
    NB = N // 16
    qn = jnp.sum(q * q, axis=1, keepdims=True)
    pn = jnp.sum(p * p, axis=0, keepdims=True)
    dot = lax.dot_general(q, p, (((1,), (0,)), ((), ())),
                          preferred_element_type=jnp.float32)
    d2 = (qn + pn) - 2.0 * dot
    fbits = lax.bitcast_convert_type(d2, jnp.int32)
    skey_ref[...] = jnp.where(fbits >= 0, fbits, fbits ^ jnp.int32(0x7FFFFFFF))
    skey0 = skey_ref[...]
    lo0 = jnp.min(skey0, axis=1, keepdims=True)
    hi0 = jnp.max(skey0, axis=1, keepdims=True)
    niota = lax.broadcasted_iota(jnp.int32, (TM, N), 1)

    def bstep(_, c):
        lo, hi = c
        mid = lo + ((hi - lo) >> 1)
        s = skey_ref[...]
        cnt = jnp.sum(jnp.where(s <= mid, 1, 0), axis=1, keepdims=True)
        ge = cnt >= K
        return jnp.where(ge, lo, mid + 1), jnp.where(ge, mid, hi)

    _, t = lax.fori_loop(0, 32, bstep, (lo0, hi0))

    s = skey_ref[...]
    c_less = jnp.sum(jnp.where(s < t, 1, 0), axis=1, keepdims=True)
    c_le = jnp.sum(jnp.where(s <= t, 1, 0), axis=1, keepdims=True)
    r = K - c_less

    def tie_fn(_):
        def jstep(_, c):
            lo, hi = c
            mid = lo + ((hi - lo) >> 1)
            s2 = skey_ref[...]
            cc = jnp.sum(jnp.where((s2 == t) & (niota <= mid), 1, 0),
                         axis=1, keepdims=True)
            ge = cc >= r
            return jnp.where(ge, lo, mid + 1), jnp.where(ge, mid, hi)
        return lax.fori_loop(0, 13, jstep,
                             (jnp.zeros((TM, 1), jnp.int32),
                              jnp.full((TM, 1), N - 1, jnp.int32)))[1]

    trigger = jnp.max(c_le) > K
    jstar = lax.cond(trigger, tie_fn,
                     lambda _: jnp.full((TM, 1), N, jnp.int32), 0)

    s = skey_ref[...]
    mask = (s < t) | ((s == t) & (niota <= jstar))
    pow2row = (1 << (lax.broadcasted_iota(jnp.int32, (1, N), 1) & 15)
               ).astype(jnp.float32)
    contrib = jnp.where(mask, pow2row, 0.0)
    bitsf = lax.dot_general(contrib.astype(jnp.bfloat16), g16_ref[...],
                            (((1,), (0,)), ((), ())),
                            preferred_element_type=jnp.float32)
    bits_ref[...] = bitsf.astype(jnp.int32)

    jiota = lax.broadcasted_iota(jnp.int32, (TM, NB), 1)
    kiota = lax.broadcasted_iota(jnp.int32, (TM, K), 1)

    def estep(k, idxl):
        bts = bits_ref[...]
        posj = jnp.min(jnp.where(bts > 0, jiota, NB), axis=1, keepdims=True)
        wsel = jnp.max(jnp.where(jiota == posj, bts, 0), axis=1, keepdims=True)
        low = wsel & (0 - wsel)
        bitidx = lax.population_count(low - 1)
        n = posj * 16 + bitidx
        bits_ref[...] = jnp.where(jiota == posj, wsel & (wsel - 1), bts)
        return jnp.where(kiota == k, n, idxl)

    idxl = lax.fori_loop(0, K, estep, jnp.zeros((TM, K), jnp.int32))
    idxc_ref[0] = idxl + b * cstride
    idxf_ref[0] = idxl + b * n_pts


def _knn_pallas(coords, coordsT, M, N, TM):
    B = coords.shape[0]
    g16 = (
        (jnp.arange(N, dtype=jnp.int32)[:, None] >> 4)
        == jnp.arange(N // 16, dtype=jnp.int32)[None, :]
    ).astype(jnp.bfloat16)
    kern = functools.partial(_knn_body, n_pts=N, cstride=_N, K=_K)
    return pl.pallas_call(
        kern,
        grid=(B, M // TM),
        in_specs=[
            pl.BlockSpec((1, TM, _COORD), lambda b, mi: (b, mi, 0)),
            pl.BlockSpec((1, _COORD, N), lambda b, mi: (b, 0, 0)),
            pl.BlockSpec((N, N // 16), lambda b, mi: (0, 0)),
        ],
        out_specs=[
            pl.BlockSpec((1, TM, _K), lambda b, mi: (b, mi, 0)),
            pl.BlockSpec((1, TM, _K), lambda b, mi: (b, mi, 0)),
        ],
        out_shape=[
            jax.ShapeDtypeStruct((B, M, _K), jnp.int32),
            jax.ShapeDtypeStruct((B, M, _K), jnp.int32),
        ],
        scratch_shapes=[
            pltpu.VMEM((TM, N), jnp.int32),
            pltpu.VMEM((TM, N // 16), jnp.int32),
        ],
    )(coords, coordsT, g16)


def _make_sc_gather(V, D, R):
    bpw = R // _NW
    mesh = plsc.VectorSubcoreMesh(core_axis_name="c", subcore_axis_name="s")

    @functools.partial(
        pl.kernel, mesh=mesh,
        out_type=jax.ShapeDtypeStruct((R, D), jnp.float32),
        compiler_params=pltpu.CompilerParams(use_tc_tiling_on_sc=False),
        scratch_types=[
            pltpu.VMEM((bpw,), jnp.int32),
            pltpu.VMEM((bpw, D), jnp.float32),
            pltpu.SemaphoreType.DMA,
        ],
    )
    def gk(table_hbm, idx_hbm, out_hbm, idx_v, rows_v, sem):
        wid = lax.axis_index("s") * 2 + lax.axis_index("c")
        base = wid * bpw
        pltpu.sync_copy(idx_hbm.at[pl.ds(base, bpw)], idx_v)
        pltpu.async_copy(table_hbm.at[idx_v], rows_v, sem).wait()
        pltpu.sync_copy(rows_v, out_hbm.at[pl.ds(base, bpw)])

    return gk


def _sc_gather(table, idx):
    V, D = table.shape
    R = idx.shape[0]
    return _make_sc_gather(V, D, R)(table, idx)


def _dense_body(gc_ref, gf_ref, q_ref, w1_ref, b1_ref, w2_ref, b2_ref,
                lw_ref, lb_ref, out_ref, *, K, C, TM):
    q = q_ref[0]
    w1 = w1_ref[...]
    b1 = b1_ref[...]
    w2 = w2_ref[...]
    b2 = b2_ref[...]
    D = _MID
    CD = C * D
    riota_c = lax.broadcasted_iota(jnp.int32, (C, CD), 0)
    ciota_c = lax.broadcasted_iota(jnp.int32, (C, CD), 1)
    E1 = (ciota_c // D == riota_c).astype(jnp.float32)
    riota_d = lax.broadcasted_iota(jnp.int32, (D, CD), 0)
    ciota_d = lax.broadcasted_iota(jnp.int32, (D, CD), 1)
    E2 = (ciota_d % D == riota_d).astype(jnp.float32)

    dn = (((1,), (0,)), ((), ()))
    agg = jnp.zeros((TM, CD), jnp.float32)
    gc = gc_ref[0]
    gf = gf_ref[0]
    for k in range(K):
        rel = gc[:, k * _CPAD:k * _CPAD + 2] - q
        h = lax.dot_general(rel, w1, dn, preferred_element_type=jnp.float32) + b1
        h = jnp.maximum(h, 0.0)
        wk = lax.dot_general(h, w2, dn, preferred_element_type=jnp.float32) + b2
        nbk = gf[:, k * C:(k + 1) * C]
        x1 = lax.dot_general(nbk, E1, dn, preferred_element_type=jnp.float32)
        x2 = lax.dot_general(wk, E2, dn, preferred_element_type=jnp.float32)
        agg = agg + x1 * x2
    out = lax.dot_general(agg, lw_ref[...], dn,
                          preferred_element_type=jnp.float32) + lb_ref[...]
    out_ref[0] = jnp.where(out >= 0.0, out, 0.2 * out)


def _dense_pallas(gc, gf, coords, w1, b1, w2, b2, lw, lb, M, C, cout, TM):
    B = coords.shape[0]
    kern = functools.partial(_dense_body, K=_K, C=C, TM=TM)
    return pl.pallas_call(
        kern,
        grid=(B, M // TM),
        in_specs=[
            pl.BlockSpec((1, TM, _K * _CPAD), lambda b, mi: (b, mi, 0)),
            pl.BlockSpec((1, TM, _K * C), lambda b, mi: (b, mi, 0)),
            pl.BlockSpec((1, TM, _COORD), lambda b, mi: (b, mi, 0)),
            pl.BlockSpec((2, _MID), lambda b, mi: (0, 0)),
            pl.BlockSpec((1, _MID), lambda b, mi: (0, 0)),
            pl.BlockSpec((_MID, _MID), lambda b, mi: (0, 0)),
            pl.BlockSpec((1, _MID), lambda b, mi: (0, 0)),
            pl.BlockSpec((C * _MID, cout), lambda b, mi: (0, 0)),
            pl.BlockSpec((1, cout), lambda b, mi: (0, 0)),
        ],
        out_specs=pl.BlockSpec((1, TM, cout), lambda b, mi: (b, mi, 0)),
        out_shape=jax.ShapeDtypeStruct((B, M, cout), jnp.float32),
    )(gc, gf, coords, w1, b1.reshape(1, -1), w2, b2.reshape(1, -1),
      lw, lb.reshape(1, -1))


def _head_body(x_ref, w1_ref, b1_ref, w2_ref, b2_ref, o_ref):
    dn = (((1,), (0,)), ((), ()))
    h = lax.dot_general(x_ref[...], w1_ref[...], dn,
                        preferred_element_type=jnp.float32) + b1_ref[...]
    h = jnp.where(h >= 0.0, h, 0.2 * h)
    o_ref[...] = lax.dot_general(h, w2_ref[...], dn,
                                 preferred_element_type=jnp.float32) + b2_ref[...]


def _head_pallas(x, fc1_w, fc1_b, fc2_w, fc2_b):
    return pl.pallas_call(
        _head_body,
        out_shape=jax.ShapeDtypeStruct((x.shape[0], fc2_w.shape[1]), jnp.float32),
    )(x, fc1_w, fc1_b.reshape(1, -1), fc2_w, fc2_b.reshape(1, -1))


def kernel(coordinates, features,
           wn1_w0, wn1_b0, wn2_w0, wn2_b0, lin_w0, lin_b0,
           wn1_w1, wn1_b1, wn2_w1, wn2_b1, lin_w1, lin_b1,
           wn1_w2, wn1_b2, wn2_w2, wn2_b2, lin_w2, lin_b2,
           fc1_w, fc1_b, fc2_w, fc2_b):
    B = coordinates.shape[0]
    coordsT = jnp.transpose(coordinates, (0, 2, 1))
    cpad = jnp.pad(coordinates, ((0, 0), (0, 0), (0, _CPAD - _COORD)))
    cpad = cpad.reshape(B * _N, _CPAD)

    idxc0, idxf0 = _knn_pallas(coordinates, coordsT, M=1024, N=8192, TM=512)
    idxc1, idxf1 = _knn_pallas(coordinates, coordsT, M=256, N=1024, TM=256)
    idxc2, idxf2 = _knn_pallas(coordinates, coordsT, M=64, N=256, TM=64)

    gc0 = _sc_gather(cpad, idxc0.reshape(-1))
    gf0 = _sc_gather(features.reshape(B * _N, _FEAT), idxf0.reshape(-1))
    f1 = _dense_pallas(gc0.reshape(B, 1024, _K * _CPAD),
                       gf0.reshape(B, 1024, _K * _FEAT),
                       coordinates, wn1_w0, wn1_b0, wn2_w0, wn2_b0,
                       lin_w0, lin_b0, M=1024, C=_FEAT, cout=32, TM=256)

    gc1 = _sc_gather(cpad, idxc1.reshape(-1))
    gf1 = _sc_gather(f1.reshape(B * 1024, 32), idxf1.reshape(-1))
    f2 = _dense_pallas(gc1.reshape(B, 256, _K * _CPAD),
                       gf1.reshape(B, 256, _K * 32),
                       coordinates, wn1_w1, wn1_b1, wn2_w1, wn2_b1,
                       lin_w1, lin_b1, M=256, C=32, cout=64, TM=256)

    gc2 = _sc_gather(cpad, idxc2.reshape(-1))
    gf2 = _sc_gather(f2.reshape(B * 256, 64), idxf2.reshape(-1))
    f3 = _dense_pallas(gc2.reshape(B, 64, _K * _CPAD),
                       gf2.reshape(B, 64, _K * 64),
                       coordinates, wn1_w2, wn1_b2, wn2_w2, wn2_b2,
                       lin_w2, lin_b2, M=64, C=64, cout=128, TM=64)

    return _head_pallas(f3.reshape(B, 64 * 128), fc1_w, fc1_b, fc2_w, fc2_b)
